# Initial kernel scaffold; baseline (speedup 1.0000x reference)
import jax
import jax.numpy as jnp
from jax.experimental import pallas as pl


def kernel(x, edge_index, W, b, g, beta):
    raise NotImplementedError("write your pallas kernel here")


# speedup vs baseline: 4.7954x; 4.7954x over previous
import functools

import jax
import jax.numpy as jnp
from jax import lax
from jax.experimental import pallas as pl
from jax.experimental.pallas import tpu as pltpu
from jax.experimental.pallas import tpu_sc as plsc

N = 10000
E = 320000
H = 128
EPS = 1e-6

NC = 2
NS = 16
NW = NC * NS
CHUNK = 128
NPAD = 10240
ROWS_PER_SUB = NPAD // NS
CPW = 80
HALF = CPW // 2
EPAD = NW * CPW * CHUNK


def _mm_body(x_ref, wt_ref, b_ref, o_ref):
    acc = jnp.dot(x_ref[...], wt_ref[...], preferred_element_type=jnp.float32)
    o_ref[...] = jnp.maximum(acc + b_ref[...], 0.0)


def _linear_relu(x, wt, b2):
    blk = 1000
    return pl.pallas_call(
        _mm_body,
        grid=(N // blk,),
        in_specs=[
            pl.BlockSpec((blk, H), lambda i: (i, 0)),
            pl.BlockSpec((H, H), lambda i: (0, 0)),
            pl.BlockSpec((1, H), lambda i: (0, 0)),
        ],
        out_specs=pl.BlockSpec((blk, H), lambda i: (i, 0)),
        out_shape=jax.ShapeDtypeStruct((N, H), jnp.float32),
    )(x, wt, b2)


def _sc_body(m_hbm, row_hbm, col_hbm, zacc_hbm, zdeg_hbm, ones_hbm,
             agg_out, deg_out,
             row_v, col_v, buf_a, buf_b, ones_v, acc_s, deg_s, sem_a, sem_b):
    c = lax.axis_index("c")
    s = lax.axis_index("s")
    wid = s * NC + c

    pltpu.sync_copy(ones_hbm, ones_v)

    r0 = s * ROWS_PER_SUB
    pltpu.sync_copy(zacc_hbm.at[pl.ds(r0, ROWS_PER_SUB)],
                    acc_s.at[pl.ds(r0, ROWS_PER_SUB)])
    pltpu.sync_copy(zdeg_hbm.at[pl.ds(r0, ROWS_PER_SUB)],
                    deg_s.at[pl.ds(r0, ROWS_PER_SUB)])
    plsc.subcore_barrier()

    bufs = (buf_a, buf_b)
    sems = (sem_a, sem_b)

    def gather(lc, t):
        pltpu.async_copy(m_hbm.at[col_v.at[lc]], bufs[t], sems[t])

    def gather_wait(lc, t):
        pltpu.make_async_copy(m_hbm.at[col_v.at[lc]], bufs[t], sems[t]).wait()

    def scatter(lc, t):
        pltpu.sync_copy(bufs[t], acc_s.at[row_v.at[lc]], add=True)
        pltpu.sync_copy(ones_v, deg_s.at[row_v.at[lc]], add=True)

    for h in range(2):
        base = wid * CPW + h * HALF
        pltpu.sync_copy(row_hbm.at[pl.ds(base, HALF)], row_v)
        pltpu.sync_copy(col_hbm.at[pl.ds(base, HALF)], col_v)

        gather(0, 0)

        def body(j, carry):
            gather(j * 2 + 1, 1)
            gather_wait(j * 2, 0)
            scatter(j * 2, 0)

            @pl.when(j < HALF // 2 - 1)
            def _():
                gather(j * 2 + 2, 0)

            gather_wait(j * 2 + 1, 1)
            scatter(j * 2 + 1, 1)
            return carry

        lax.fori_loop(0, HALF // 2, body, 0)

    plsc.subcore_barrier()
    pltpu.sync_copy(acc_s.at[pl.ds(r0, ROWS_PER_SUB)],
                    agg_out.at[c, pl.ds(r0, ROWS_PER_SUB)])
    pltpu.sync_copy(deg_s.at[pl.ds(r0, ROWS_PER_SUB)],
                    deg_out.at[pl.ds(c * NPAD + r0, ROWS_PER_SUB)])


_sc_aggregate = functools.partial(
    pl.kernel,
    out_type=(
        jax.ShapeDtypeStruct((NC, NPAD, H), jnp.float32),
        jax.ShapeDtypeStruct((NC * NPAD,), jnp.float32),
    ),
    mesh=plsc.VectorSubcoreMesh(core_axis_name="c", subcore_axis_name="s"),
    scratch_types=[
        pltpu.VMEM((HALF, CHUNK), jnp.int32),
        pltpu.VMEM((HALF, CHUNK), jnp.int32),
        pltpu.VMEM((CHUNK, H), jnp.float32),
        pltpu.VMEM((CHUNK, H), jnp.float32),
        pltpu.VMEM((CHUNK,), jnp.float32),
        pltpu.VMEM_SHARED((NPAD, H), jnp.float32),
        pltpu.VMEM_SHARED((NPAD,), jnp.float32),
        pltpu.SemaphoreType.DMA,
        pltpu.SemaphoreType.DMA,
    ],
)(_sc_body)


def _fin_body(x_ref, a0_ref, a1_ref, d0_ref, d1_ref, g_ref, beta_ref, o_ref):
    agg = a0_ref[...] + a1_ref[...]
    deg = d0_ref[...] + d1_ref[...]
    msg = agg / jnp.where(deg == 0.0, 1.0, deg)
    h = x_ref[...] + msg
    rms = jnp.sqrt(jnp.mean(h * h, axis=1, keepdims=True) + EPS)
    o_ref[...] = (h / rms) * g_ref[...] + beta_ref[...]


def _finalize(x, a0, a1, d0, d1, g2, beta2):
    blk = 1000
    return pl.pallas_call(
        _fin_body,
        grid=(N // blk,),
        in_specs=[
            pl.BlockSpec((blk, H), lambda i: (i, 0)),
            pl.BlockSpec((blk, H), lambda i: (i, 0)),
            pl.BlockSpec((blk, H), lambda i: (i, 0)),
            pl.BlockSpec((blk, 1), lambda i: (i, 0)),
            pl.BlockSpec((blk, 1), lambda i: (i, 0)),
            pl.BlockSpec((1, H), lambda i: (0, 0)),
            pl.BlockSpec((1, H), lambda i: (0, 0)),
        ],
        out_specs=pl.BlockSpec((blk, H), lambda i: (i, 0)),
        out_shape=jax.ShapeDtypeStruct((N, H), jnp.float32),
    )(x, a0, a1, d0, d1, g2, beta2)


def kernel(x, edge_index, W, b, g, beta):
    m = _linear_relu(x, W.T, b.reshape(1, H))

    row = edge_index[0]
    col = edge_index[1]
    npad_e = EPAD - E
    row_p = jnp.concatenate(
        [row, jnp.full((npad_e,), N, dtype=jnp.int32)]
    ).reshape(CPW, NW, CHUNK).transpose(1, 0, 2).reshape(NW * CPW, CHUNK)
    col_p = jnp.concatenate(
        [col, jnp.zeros((npad_e,), dtype=jnp.int32)]
    ).reshape(CPW, NW, CHUNK).transpose(1, 0, 2).reshape(NW * CPW, CHUNK)

    zacc = jnp.zeros((NPAD, H), dtype=jnp.float32)
    zdeg = jnp.zeros((NPAD,), dtype=jnp.float32)
    ones = jnp.ones((CHUNK,), dtype=jnp.float32)

    agg2, deg2 = _sc_aggregate(m, row_p, col_p, zacc, zdeg, ones)

    a0 = agg2[0, :N]
    a1 = agg2[1, :N]
    degs = deg2.reshape(NC, NPAD)
    d0 = degs[0, :N].reshape(N, 1)
    d1 = degs[1, :N].reshape(N, 1)

    return _finalize(x, a0, a1, d0, d1, g.reshape(1, H), beta.reshape(1, H))
